# baseline (speedup 1.0000x reference)
import functools

import jax
import jax.numpy as jnp
from jax import lax
from jax.experimental import pallas as pl
from jax.experimental.pallas import tpu as pltpu
from jax.experimental.pallas import tpu_sc as plsc
from jax._src.pallas import mpmd

_N = 128
_P = 64
_B = 4096
_NN = _N * _N
_EPS = 1e-06


def _table_body(s_ref, g_ref, m_ref):
    s = s_ref[...]
    g = g_ref[...]
    row = lax.broadcasted_iota(jnp.int32, (_N, _N), 0)
    col = lax.broadcasted_iota(jnp.int32, (_N, _N), 1)
    offdiag = (row != col).astype(s.dtype)
    sz = s * offdiag[None, :, :]
    denom = jnp.maximum(jnp.sum(jnp.abs(sz), axis=-1, keepdims=True), _EPS)
    sp = jnp.maximum(g, 0.0) + jnp.log1p(jnp.exp(-jnp.abs(g))) + 1e-06
    sp = sp * (_N / jnp.maximum(jnp.sum(sp, axis=-1, keepdims=True), _EPS))
    m_ref[...] = (sz / denom) * sp[:, :, None]


def _build_table(S, G):
    return pl.pallas_call(
        _table_body,
        out_shape=jax.ShapeDtypeStruct((_P, _N, _N), jnp.float32),
    )(S, G)


_NSCS = 2
_NTILE = 16
_NW = _NSCS * _NTILE

_XS = 2304
_BPS = _XS // _NSCS
_IDXCH = 128
_NREF = _BPS // _IDXCH
_LAG = 32

_BT = _B - _XS
_TPW = _BT // _NW


def _scs_fn(table_hbm, idxa_hbm, idxb_hbm, out_hbm,
            idx_s, spt, idx_v, rows_v, semt, sem0, semg0, semg1):
    del idxb_hbm, idx_v, rows_v, semg0, semg1
    cid = lax.axis_index("c")
    base = cid * _BPS
    tcopy = pltpu.async_copy(table_hbm, spt, semt)

    def drain_one():
        pltpu.make_async_copy(spt.at[0], out_hbm.at[base], sem0).wait()

    def refill(r, carry):
        pltpu.sync_copy(idxa_hbm.at[pl.ds(base + r * _IDXCH, _IDXCH)], idx_s)

        def body(j, carry2):
            i = r * _IDXCH + j
            pltpu.async_copy(spt.at[idx_s[j]], out_hbm.at[base + i], sem0)

            @pl.when(i >= _LAG)
            def _():
                drain_one()

            return carry2

        lax.fori_loop(0, _IDXCH, body, carry)
        return carry

    tcopy.wait()
    lax.fori_loop(0, _NREF, refill, 0)
    for _ in range(_LAG):
        drain_one()


def _tec_fn(table_hbm, idxa_hbm, idxb_hbm, out_hbm,
            idx_s, spt, idx_v, rows_v, semt, sem0, semg0, semg1):
    del idxa_hbm, idx_s, spt, semt, sem0
    wid = lax.axis_index("s") * _NSCS + lax.axis_index("c")
    base = _XS + wid * _TPW
    cbase = wid * _TPW
    pltpu.sync_copy(idxb_hbm.at[pl.ds(cbase, _TPW)], idx_v)

    def body(i, carry):
        c0 = i * 2
        g0 = pltpu.async_copy(table_hbm.at[idx_v.at[c0]], rows_v.at[0], semg0)
        g1 = pltpu.async_copy(table_hbm.at[idx_v.at[c0 + 1]], rows_v.at[1], semg1)
        g0.wait()
        pltpu.sync_copy(rows_v.at[0], out_hbm.at[pl.ds(base + c0, 1)])
        g1.wait()
        pltpu.sync_copy(rows_v.at[1], out_hbm.at[pl.ds(base + c0 + 1, 1)])
        return carry

    lax.fori_loop(0, _TPW // 2, body, 0)


@jax.jit
def _gather(table, idxa, idxb):
    scalar_mesh = plsc.ScalarSubcoreMesh(axis_name="c", num_cores=_NSCS)
    vector_mesh = plsc.VectorSubcoreMesh(core_axis_name="c", subcore_axis_name="s")
    f = mpmd.mpmd_map(
        [(scalar_mesh, _scs_fn), (vector_mesh, _tec_fn)],
        out_types=jax.ShapeDtypeStruct((_B, _NN), jnp.float32),
        scratch_types=[
            (pltpu.SMEM @ scalar_mesh)((_IDXCH,), jnp.int32),
            pltpu.VMEM_SHARED((_P, _NN), jnp.float32),
            (pltpu.VMEM @ vector_mesh)((_TPW, 1), jnp.int32),
            (pltpu.VMEM @ vector_mesh)((2, 1, _NN), jnp.float32),
            pltpu.SemaphoreType.DMA @ scalar_mesh,
            pltpu.SemaphoreType.DMA @ scalar_mesh,
            pltpu.SemaphoreType.DMA @ vector_mesh,
            pltpu.SemaphoreType.DMA @ vector_mesh,
        ],
    )
    return f(table, idxa, idxb)


def kernel(phases, S, G):
    table = _build_table(S.astype(jnp.float32), G.astype(jnp.float32))
    table = table.reshape(_P, _NN)
    ph = phases.astype(jnp.int32)
    out = _gather(table, ph[:_XS], ph[_XS:].reshape(_BT, 1))
    return out.reshape(_B, _N, _N)


# speedup vs baseline: 1.0009x; 1.0009x over previous
import functools

import jax
import jax.numpy as jnp
from jax import lax
from jax.experimental import pallas as pl
from jax.experimental.pallas import tpu as pltpu
from jax.experimental.pallas import tpu_sc as plsc
from jax._src.pallas import mpmd

_N = 128
_P = 64
_B = 4096
_NN = _N * _N
_EPS = 1e-06


def _table_body(s_ref, g_ref, m_ref):
    s = s_ref[...]
    g = g_ref[...]
    row = lax.broadcasted_iota(jnp.int32, (_N, _N), 0)
    col = lax.broadcasted_iota(jnp.int32, (_N, _N), 1)
    offdiag = (row != col).astype(s.dtype)
    sz = s * offdiag[None, :, :]
    denom = jnp.maximum(jnp.sum(jnp.abs(sz), axis=-1, keepdims=True), _EPS)
    sp = jnp.maximum(g, 0.0) + jnp.log1p(jnp.exp(-jnp.abs(g))) + 1e-06
    sp = sp * (_N / jnp.maximum(jnp.sum(sp, axis=-1, keepdims=True), _EPS))
    m_ref[...] = (sz / denom) * sp[:, :, None]


def _build_table(S, G):
    return pl.pallas_call(
        _table_body,
        out_shape=jax.ShapeDtypeStruct((_P, _N, _N), jnp.float32),
    )(S, G)


_NSCS = 2
_NTILE = 16
_NW = _NSCS * _NTILE

_XS = 2304
_BPS = _XS // _NSCS
_IDXCH = 128
_NREF = _BPS // _IDXCH
_LAG = 32

_BT = _B - _XS
_TPW = _BT // _NW


def _scs_fn(table_hbm, idxa_hbm, idxb_hbm, out_hbm,
            idx_s, spt, idx_v, rows_v, semt, sem0, semg0, semg1):
    del idxb_hbm, idx_v, rows_v, semg0, semg1
    cid = lax.axis_index("c")
    base = cid * _BPS
    tcopy = pltpu.async_copy(table_hbm, spt, semt)

    def drain_one():
        pltpu.make_async_copy(spt.at[0], out_hbm.at[base], sem0).wait()

    def refill(r, carry):
        pltpu.sync_copy(idxa_hbm.at[pl.ds(base + r * _IDXCH, _IDXCH)], idx_s)

        def body(j, carry2):
            i = r * _IDXCH + j
            pltpu.async_copy(spt.at[idx_s[j]], out_hbm.at[base + i], sem0)

            @pl.when(i >= _LAG)
            def _():
                drain_one()

            return carry2

        lax.fori_loop(0, _IDXCH, body, carry)
        return carry

    tcopy.wait()
    lax.fori_loop(0, _NREF, refill, 0)
    for _ in range(_LAG):
        drain_one()


def _tec_fn(table_hbm, idxa_hbm, idxb_hbm, out_hbm,
            idx_s, spt, idx_v, rows_v, semt, sem0, semg0, semg1):
    del idxa_hbm, idx_s, spt, semt, sem0
    wid = lax.axis_index("s") * _NSCS + lax.axis_index("c")
    base = _XS + wid * _TPW
    cbase = wid * _TPW
    pltpu.sync_copy(idxb_hbm.at[pl.ds(cbase, _TPW)], idx_v)

    def body(i, carry):
        c0 = i * 2
        g0 = pltpu.async_copy(table_hbm.at[idx_v.at[c0]], rows_v.at[0], semg0)
        g1 = pltpu.async_copy(table_hbm.at[idx_v.at[c0 + 1]], rows_v.at[1], semg1)
        g0.wait()
        pltpu.sync_copy(rows_v.at[0], out_hbm.at[pl.ds(base + c0, 1)])
        g1.wait()
        pltpu.sync_copy(rows_v.at[1], out_hbm.at[pl.ds(base + c0 + 1, 1)])
        return carry

    lax.fori_loop(0, _TPW // 2, body, 0)


@jax.jit
def _gather(table, idxa, idxb):
    scalar_mesh = plsc.ScalarSubcoreMesh(axis_name="c", num_cores=_NSCS)
    vector_mesh = plsc.VectorSubcoreMesh(core_axis_name="c", subcore_axis_name="s")
    f = mpmd.mpmd_map(
        [(vector_mesh, _tec_fn), (scalar_mesh, _scs_fn)],
        out_types=jax.ShapeDtypeStruct((_B, _NN), jnp.float32),
        scratch_types=[
            (pltpu.SMEM @ scalar_mesh)((_IDXCH,), jnp.int32),
            pltpu.VMEM_SHARED((_P, _NN), jnp.float32),
            (pltpu.VMEM @ vector_mesh)((_TPW, 1), jnp.int32),
            (pltpu.VMEM @ vector_mesh)((2, 1, _NN), jnp.float32),
            pltpu.SemaphoreType.DMA @ scalar_mesh,
            pltpu.SemaphoreType.DMA @ scalar_mesh,
            pltpu.SemaphoreType.DMA @ vector_mesh,
            pltpu.SemaphoreType.DMA @ vector_mesh,
        ],
    )
    return f(table, idxa, idxb)


def kernel(phases, S, G):
    table = _build_table(S.astype(jnp.float32), G.astype(jnp.float32))
    table = table.reshape(_P, _NN)
    ph = phases.astype(jnp.int32)
    out = _gather(table, ph[:_XS], ph[_XS:].reshape(_BT, 1))
    return out.reshape(_B, _N, _N)
